# baseline (speedup 1.0000x reference)
import functools
import jax
import jax.numpy as jnp
from jax import lax
from jax.experimental import pallas as pl
from jax.experimental.pallas import tpu as pltpu
from jax.experimental.pallas import tpu_sc as plsc

_N = 10000
_LAM = 1.0
_B = 2
_NBLK = _N // _B
_NW = 32
_CHUNKS = _N // 16
_NBUF = 6

_FULL = _NBLK // _NW
_REM = _NBLK % _NW
_T = -(-(_FULL + (1 if _REM else 0)) // _NBUF) * _NBUF


def _process_block(buf, p, r0):
    invs = []
    for b in range(_B):
        def sum_body(j, accs):
            base = j * 80
            return tuple(accs[k] + buf[p, b, pl.ds(base + k * 16, 16)]
                         for k in range(5))

        accs = lax.fori_loop(
            0, _CHUNKS // 5, sum_body,
            tuple(jnp.zeros((16,), jnp.float32) for _ in range(5)), unroll=2)
        acc = (accs[0] + accs[1]) + (accs[2] + accs[3]) + accs[4]
        rs = jnp.sum(acc)
        den_v = jnp.full((16,), rs + 1.0, jnp.float32)
        invs.append(jnp.where(den_v == 0.0, 0.0, 1.0 / den_v))

    for b in range(_B):
        inv_v = invs[b]

        def scale_body(j, carry):
            sl = pl.ds(j * 16, 16)
            buf[p, b, sl] = buf[p, b, sl] * inv_v
            return carry

        lax.fori_loop(0, _CHUNKS, scale_body, 0, unroll=8)

    for b in range(_B):
        r = r0 + b
        jd = r // 16
        lane = r % 16
        sl = pl.ds(jd * 16, 16)
        v = buf[p, b, sl]
        m = (lax.iota(jnp.int32, 16) == lane).astype(jnp.float32)
        buf[p, b, sl] = v + m * (_LAM * v + (1.0 + _LAM) * invs[b])


def _sc_body(a_hbm, out_hbm, buf, *sems):
    insems = sems[:_NBUF]
    outsems = sems[_NBUF:]
    c = lax.axis_index("c")
    s = lax.axis_index("s")
    wid = s * 2 + c
    tlast = _FULL - 1 + jnp.where(wid < _REM, 1, 0)

    def blk_of(t):
        return wid + _NW * jnp.minimum(t, tlast)

    pltpu.async_copy(a_hbm.at[pl.ds(blk_of(0) * _B, _B)], buf.at[0], insems[0])

    @pl.loop(0, _T, step=_NBUF)
    def _steps(t0):
        for p in range(_NBUF):
            t = t0 + p
            pn = (p + 1) % _NBUF
            r0 = blk_of(t) * _B
            pltpu.make_async_copy(
                a_hbm.at[pl.ds(r0, _B)], buf.at[p], insems[p]).wait()

            @pl.when(t >= _NBUF - 1)
            def _():
                rprev = blk_of(t - (_NBUF - 1)) * _B
                pltpu.make_async_copy(
                    buf.at[pn], out_hbm.at[pl.ds(rprev, _B)],
                    outsems[pn]).wait()

            @pl.when(t + 1 < _T)
            def _():
                rnext = blk_of(t + 1) * _B
                pltpu.async_copy(
                    a_hbm.at[pl.ds(rnext, _B)], buf.at[pn], insems[pn])

            _process_block(buf, p, r0)
            pltpu.async_copy(buf.at[p], out_hbm.at[pl.ds(r0, _B)], outsems[p])

    for t in range(_T - (_NBUF - 1), _T):
        p = t % _NBUF
        r0 = blk_of(t) * _B
        pltpu.make_async_copy(
            buf.at[p], out_hbm.at[pl.ds(r0, _B)], outsems[p]).wait()


def kernel(adjacency):
    adjacency = adjacency.astype(jnp.float32)
    f = functools.partial(
        pl.kernel,
        out_type=jax.ShapeDtypeStruct((_N, _N), jnp.float32),
        mesh=plsc.VectorSubcoreMesh(core_axis_name="c", subcore_axis_name="s"),
        scratch_types=[pltpu.VMEM((_NBUF, _B, _N), jnp.float32)]
        + [pltpu.SemaphoreType.DMA] * (2 * _NBUF),
        compiler_params=pltpu.CompilerParams(needs_layout_passes=False),
    )(_sc_body)
    return f(adjacency)


# speedup vs baseline: 1.0284x; 1.0284x over previous
import functools
import jax
import jax.numpy as jnp
from jax import lax
from jax.experimental import pallas as pl
from jax.experimental.pallas import tpu as pltpu
from jax.experimental.pallas import tpu_sc as plsc

_N = 10000
_LAM = 1.0
_B = 4
_NBLK = _N // _B
_NW = 32
_CHUNKS = _N // 16
_NBUF = 3

_FULL = _NBLK // _NW
_REM = _NBLK % _NW
_T = -(-(_FULL + (1 if _REM else 0)) // _NBUF) * _NBUF


def _process_block(buf, p, r0):
    invs = []
    for b in range(_B):
        def sum_body(j, accs):
            base = j * 80
            return tuple(accs[k] + buf[p, b, pl.ds(base + k * 16, 16)]
                         for k in range(5))

        accs = lax.fori_loop(
            0, _CHUNKS // 5, sum_body,
            tuple(jnp.zeros((16,), jnp.float32) for _ in range(5)), unroll=2)
        acc = (accs[0] + accs[1]) + (accs[2] + accs[3]) + accs[4]
        rs = jnp.sum(acc)
        den_v = jnp.full((16,), rs + 1.0, jnp.float32)
        invs.append(jnp.where(den_v == 0.0, 0.0, 1.0 / den_v))

    for b in range(_B):
        inv_v = invs[b]

        @plsc.parallel_loop(0, _CHUNKS, unroll=8)
        def scale_body(j):
            sl = pl.ds(j * 16, 16)
            buf[p, b, sl] = buf[p, b, sl] * inv_v

    for b in range(_B):
        r = r0 + b
        jd = r // 16
        lane = r % 16
        sl = pl.ds(jd * 16, 16)
        v = buf[p, b, sl]
        m = (lax.iota(jnp.int32, 16) == lane).astype(jnp.float32)
        buf[p, b, sl] = v + m * (_LAM * v + (1.0 + _LAM) * invs[b])


def _sc_body(a_hbm, out_hbm, buf, *sems):
    insems = sems[:_NBUF]
    outsems = sems[_NBUF:]
    c = lax.axis_index("c")
    s = lax.axis_index("s")
    wid = s * 2 + c
    tlast = _FULL - 1 + jnp.where(wid < _REM, 1, 0)

    def blk_of(t):
        return wid + _NW * jnp.minimum(t, tlast)

    pltpu.async_copy(a_hbm.at[pl.ds(blk_of(0) * _B, _B)], buf.at[0], insems[0])

    @pl.loop(0, _T, step=_NBUF)
    def _steps(t0):
        for p in range(_NBUF):
            t = t0 + p
            pn = (p + 1) % _NBUF
            r0 = blk_of(t) * _B
            pltpu.make_async_copy(
                a_hbm.at[pl.ds(r0, _B)], buf.at[p], insems[p]).wait()

            @pl.when(t >= _NBUF - 1)
            def _():
                rprev = blk_of(t - (_NBUF - 1)) * _B
                pltpu.make_async_copy(
                    buf.at[pn], out_hbm.at[pl.ds(rprev, _B)],
                    outsems[pn]).wait()

            @pl.when(t + 1 < _T)
            def _():
                rnext = blk_of(t + 1) * _B
                pltpu.async_copy(
                    a_hbm.at[pl.ds(rnext, _B)], buf.at[pn], insems[pn])

            _process_block(buf, p, r0)
            pltpu.async_copy(buf.at[p], out_hbm.at[pl.ds(r0, _B)], outsems[p])

    for t in range(_T - (_NBUF - 1), _T):
        p = t % _NBUF
        r0 = blk_of(t) * _B
        pltpu.make_async_copy(
            buf.at[p], out_hbm.at[pl.ds(r0, _B)], outsems[p]).wait()


def kernel(adjacency):
    adjacency = adjacency.astype(jnp.float32)
    f = functools.partial(
        pl.kernel,
        out_type=jax.ShapeDtypeStruct((_N, _N), jnp.float32),
        mesh=plsc.VectorSubcoreMesh(core_axis_name="c", subcore_axis_name="s"),
        scratch_types=[pltpu.VMEM((_NBUF, _B, _N), jnp.float32)]
        + [pltpu.SemaphoreType.DMA] * (2 * _NBUF),
        compiler_params=pltpu.CompilerParams(needs_layout_passes=False),
    )(_sc_body)
    return f(adjacency)
